# baseline (speedup 1.0000x reference)
import functools

import jax
import jax.numpy as jnp
from jax import lax
from jax.experimental import pallas as pl
from jax.experimental.pallas import tpu as pltpu
from jax.experimental.pallas import tpu_sc as plsc

_B, _N, _C, _D = 8, 2048, 3, 64
_NP, _NS = 512, 32
_CD = _C + _D
_TW = 128
_NW = 32
_QPW = (_B * _NP) // _NW
_CH = 8
_NCHUNK = _QPW // _CH
_NK = _N // 16


def _fps_body(xt_ref, c0_ref, out_ref):
    x = xt_ref[0]
    y = xt_ref[1]
    z = xt_ref[2]
    iota = lax.broadcasted_iota(jnp.int32, (_B, _N), 1)
    iota_np = lax.broadcasted_iota(jnp.int32, (_B, _NP), 1)
    dist0 = jnp.full((_B, _N), 1e10, jnp.float32)
    acc0 = jnp.zeros((_B, _NP), jnp.float32)
    xyz3 = jnp.concatenate([x, y, z], axis=0)
    ones = jnp.ones((_B, _N), jnp.float32)
    xyz3o = jnp.concatenate([xyz3, ones], axis=0)
    onescol = jnp.ones((_N, 1), jnp.float32)

    def step(t, carry):
        distance, cx, cy, cz, ax, ay, az = carry
        sel = iota_np == t
        ax = jnp.where(sel, cx, ax)
        ay = jnp.where(sel, cy, ay)
        az = jnp.where(sel, cz, az)
        dx = x - cx
        dy = y - cy
        dz = z - cz
        d = dx * dx + dy * dy + dz * dz
        distance = jnp.minimum(distance, d)
        m = jnp.max(distance, axis=1, keepdims=True)
        eq = distance == m
        eq4 = jnp.concatenate([eq, eq, eq, eq], axis=0)
        w = jnp.where(eq4, xyz3o, 0.0)
        sums = jax.lax.dot_general(w, onescol, (((1,), (0,)), ((), ())),
                                   preferred_element_type=jnp.float32)
        tie = jnp.max(sums[3 * _B:4 * _B])

        def no_tie(_):
            return sums[0:_B], sums[_B:2 * _B], sums[2 * _B:3 * _B]

        def with_tie(_):
            fidx = jnp.min(jnp.where(eq, iota, _N), axis=1, keepdims=True)
            oh = iota == fidx
            oh3 = jnp.concatenate([oh, oh, oh], axis=0)
            s2 = jnp.sum(jnp.where(oh3, xyz3, 0.0), axis=1, keepdims=True)
            return s2[0:_B], s2[_B:2 * _B], s2[2 * _B:3 * _B]

        ncx, ncy, ncz = lax.cond(tie <= 1.0, no_tie, with_tie, 0)
        return distance, ncx, ncy, ncz, ax, ay, az

    _, _, _, _, ax, ay, az = lax.fori_loop(
        0, _NP, step, (dist0, c0_ref[0], c0_ref[1], c0_ref[2],
                       acc0, acc0, acc0))
    out_ref[0] = ax
    out_ref[1] = ay
    out_ref[2] = az


def _fps(xt, c0):
    return pl.pallas_call(
        _fps_body,
        out_shape=jax.ShapeDtypeStruct((_C, _B, _NP), jnp.float32),
    )(xt, c0)


def _sqdist_body(xt_ref, q_ref, r2_ref, out_ref, kst_ref):
    xm = xt_ref[0]
    q = q_ref[0]
    dot = jax.lax.dot_general(q, xm, (((1,), (0,)), ((), ())),
                              preferred_element_type=jnp.float32)
    d = -2.0 * dot
    qx = q[:, 0:1]
    qy = q[:, 1:2]
    qz = q[:, 2:3]
    qs = (qx * qx + qy * qy) + qz * qz
    x = xm[0:1, :]
    y = xm[1:2, :]
    z = xm[2:3, :]
    ns = (x * x + y * y) + z * z
    d = d + qs
    d = d + ns
    out_ref[0] = d
    r2 = r2_ref[0]
    maskf = jnp.where(d <= r2, 1.0, 0.0)
    sel_n = lax.broadcasted_iota(jnp.int32, (_N, _NK), 0) // 16
    sel_k = lax.broadcasted_iota(jnp.int32, (_N, _NK), 1)
    sel = jnp.where(sel_n <= sel_k, 1.0, 0.0)
    cum = jax.lax.dot_general(maskf, sel, (((1,), (0,)), ((), ())),
                              preferred_element_type=jnp.float32)
    nfull = jnp.sum(jnp.where(cum < float(_NS), 1.0, 0.0), axis=1,
                    keepdims=True)
    kst = jnp.minimum(nfull + 1.0, float(_NK)).astype(jnp.int32)
    kst_ref[0] = jnp.broadcast_to(kst, (_NP, 16))


def _sqdist(xt, newxyz_t, r2s):
    return pl.pallas_call(
        _sqdist_body,
        grid=(_B,),
        in_specs=[
            pl.BlockSpec((1, _C, _N), lambda b: (b, 0, 0)),
            pl.BlockSpec((1, _NP, _C), lambda b: (b, 0, 0)),
            pl.BlockSpec(memory_space=pltpu.SMEM),
        ],
        out_specs=[
            pl.BlockSpec((1, _NP, _N), lambda b: (b, 0, 0)),
            pl.BlockSpec((1, _NP, 16), lambda b: (b, 0, 0)),
        ],
        out_shape=[
            jax.ShapeDtypeStruct((_B, _NP, _N), jnp.float32),
            jax.ShapeDtypeStruct((_B, _NP, 16), jnp.int32),
        ],
    )(xt, newxyz_t, r2s)


def _group_body(dists_hbm, tbl_hbm, corr_hbm, rad2_hbm, kst_hbm, out_hbm,
                drow0, drow1, selb, idxb, rows0, rows1, corrv, rad2v,
                kstv0, kstv1, semd, semg, semo0, semo1):
    c = lax.axis_index("c")
    s = lax.axis_index("s")
    wid = s * 2 + c
    qbase = wid * _QPW
    b = qbase // _NP
    base = (b * _N).astype(jnp.int32)
    pltpu.sync_copy(corr_hbm.at[pl.ds(qbase, _QPW)], corrv)
    pltpu.sync_copy(rad2_hbm, rad2v)
    rad2 = rad2v[...]
    iota16 = lax.iota(jnp.int32, 16)
    bufs = ((drow0, kstv0, rows0, semo0), (drow1, kstv1, rows1, semo1))

    def prefetch(ci, dr, kv):
        q0 = qbase + ci * _CH
        pltpu.async_copy(dists_hbm.at[pl.ds(q0, _CH)], dr, semd)
        pltpu.async_copy(kst_hbm.at[pl.ds(q0, _CH)], kv, semd)

    prefetch(0, drow0, kstv0)
    prefetch(1, drow1, kstv1)

    def sup_body(sup, _):
        for b2, (dr, kv, rw, so) in enumerate(bufs):
            ci = sup * 2 + b2
            q0 = qbase + ci * _CH
            pltpu.make_async_copy(dists_hbm.at[pl.ds(q0, _CH)], dr,
                                  semd).wait()
            pltpu.make_async_copy(kst_hbm.at[pl.ds(q0, _CH)], kv,
                                  semd).wait()
            @pl.when(sup > 0)
            def _():
                pltpu.make_async_copy(
                    rw, out_hbm.at[pl.ds(q0 * _NS, _CH * _NS)], so).wait()

            copies = []
            for j in range(_CH):
                def body(k, cnt, j=j, dr=dr):
                    v0 = dr[j, pl.ds(k * 32, 16)]
                    v1 = dr[j, pl.ds(k * 32 + 16, 16)]
                    m0 = v0 <= rad2
                    m1 = v1 <= rad2
                    c0 = plsc.cumsum(jnp.where(m0, 1, 0))
                    c1 = plsc.cumsum(jnp.where(m1, 1, 0))
                    pc0 = plsc.all_reduce_population_count(m0)
                    pc1 = plsc.all_reduce_population_count(m1)
                    plsc.store_scatter(selb, [cnt + c0 - 1],
                                       iota16 + k * 32, mask=m0)
                    cnt1 = cnt + pc0[0]
                    plsc.store_scatter(selb, [cnt1 + c1 - 1],
                                       iota16 + (k * 32 + 16), mask=m1)
                    return cnt1 + pc1[0]

                ks = kv[j, pl.ds(0, 16)][0]
                cnt = lax.fori_loop(0, (ks + 1) >> 1, body, jnp.int32(0))
                s0 = selb[pl.ds(0, 16)]
                first = s0[0]
                s1 = selb[pl.ds(16, 16)]
                fv = jnp.zeros((16,), jnp.int32) + first
                p0 = jnp.where(iota16 < cnt, s0, fv) + base
                p1 = jnp.where(iota16 + 16 < cnt, s1, fv) + base
                idxb[j, pl.ds(0, 16)] = p0
                idxb[j, pl.ds(16, 16)] = p1
                copies.append(
                    pltpu.async_copy(tbl_hbm.at[idxb.at[j]],
                                     rw.at[pl.ds(j * _NS, _NS)], semg))
            for j in range(_CH):
                copies[j].wait()
                cv = corrv[ci * _CH + j, pl.ds(0, 16)]

                def fix_row(r, _, j=j, cv=cv, rw=rw):
                    rr = j * _NS + r
                    rw[rr, pl.ds(0, 16)] = rw[rr, pl.ds(0, 16)] - cv
                    return 0

                lax.fori_loop(0, _NS, fix_row, 0)
            pltpu.async_copy(rw, out_hbm.at[pl.ds(q0 * _NS, _CH * _NS)],
                             so)

            @pl.when(sup < _NCHUNK // 2 - 1)
            def _():
                prefetch(ci + 2, dr, kv)
        return 0

    lax.fori_loop(0, _NCHUNK // 2, sup_body, 0)
    for rw, so in ((rows0, semo0), (rows1, semo1)):
        pltpu.make_async_copy(rw, out_hbm.at[pl.ds(0, _CH * _NS)],
                              so).wait()


def _group(dists, tbl, corr, rad2, ksts):
    mesh = plsc.VectorSubcoreMesh(core_axis_name="c", subcore_axis_name="s")
    f = functools.partial(
        pl.kernel,
        out_type=jax.ShapeDtypeStruct((_B * _NP * _NS, _TW), jnp.float32),
        mesh=mesh,
        compiler_params=pltpu.CompilerParams(needs_layout_passes=False),
        scratch_types=[
            pltpu.VMEM((_CH, _N), jnp.float32),
            pltpu.VMEM((_CH, _N), jnp.float32),
            pltpu.VMEM((80,), jnp.int32),
            pltpu.VMEM((_CH, _NS), jnp.int32),
            pltpu.VMEM((_CH * _NS, _TW), jnp.float32),
            pltpu.VMEM((_CH * _NS, _TW), jnp.float32),
            pltpu.VMEM((_QPW, 16), jnp.float32),
            pltpu.VMEM((16,), jnp.float32),
            pltpu.VMEM((_CH, 16), jnp.int32),
            pltpu.VMEM((_CH, 16), jnp.int32),
            pltpu.SemaphoreType.DMA,
            pltpu.SemaphoreType.DMA,
            pltpu.SemaphoreType.DMA,
            pltpu.SemaphoreType.DMA,
        ],
    )(_group_body)
    return f(dists, tbl, corr, rad2, ksts)


def kernel(xyz, points, npoint, radius, nsample):
    del nsample
    f0 = jnp.asarray(npoint, jnp.int32) - _NP
    xt = jnp.transpose(xyz, (2, 0, 1))
    c0 = jnp.transpose(xyz[jnp.arange(_B), f0], (1, 0))[:, :, None]
    newxyz_t = _fps(xt, c0)
    new_xyz = jnp.transpose(newxyz_t, (1, 2, 0))
    r2 = (radius * radius).astype(jnp.float32)
    dists, ksts = _sqdist(jnp.transpose(xt, (1, 0, 2)), new_xyz,
                          r2.reshape(1))
    tbl = jnp.pad(
        jnp.concatenate([xyz, points], axis=-1).reshape(_B * _N, _CD),
        ((0, 0), (0, _TW - _CD)))
    corr = jnp.pad(new_xyz.reshape(_B * _NP, _C), ((0, 0), (0, 16 - _C)))
    rad2 = jnp.full((16,), r2, jnp.float32)
    out = _group(dists.reshape(_B * _NP, _N), tbl, corr, rad2,
                 ksts.reshape(_B * _NP, 16))
    new_points = out[:, :_CD].reshape(_B, _NP, _NS, _CD)
    return new_xyz, new_points


# speedup vs baseline: 1.1990x; 1.1990x over previous
import functools

import jax
import jax.numpy as jnp
from jax import lax
from jax.experimental import pallas as pl
from jax.experimental.pallas import tpu as pltpu
from jax.experimental.pallas import tpu_sc as plsc

_B, _N, _C, _D = 8, 2048, 3, 64
_NP, _NS = 512, 32
_CD = _C + _D
_TW = 128
_NW = 32
_QPW = (_B * _NP) // _NW
_CH = 8
_NCHUNK = _QPW // _CH
_NK = _N // 16


def _fps_body(xt_ref, c0_ref, out_ref):
    x = xt_ref[0]
    y = xt_ref[1]
    z = xt_ref[2]
    iota = lax.broadcasted_iota(jnp.int32, (_B, _N), 1)
    iota_np = lax.broadcasted_iota(jnp.int32, (_B, _NP), 1)
    dist0 = jnp.full((_B, _N), 1e10, jnp.float32)
    acc0 = jnp.zeros((_B, _NP), jnp.float32)
    xyz3 = jnp.concatenate([x, y, z], axis=0)
    ones = jnp.ones((_B, _N), jnp.float32)
    xyz3o = jnp.concatenate([xyz3, ones], axis=0)

    def step(t, carry):
        distance, cx, cy, cz, ax, ay, az = carry
        sel = iota_np == t
        ax = jnp.where(sel, cx, ax)
        ay = jnp.where(sel, cy, ay)
        az = jnp.where(sel, cz, az)
        dx = x - cx
        dy = y - cy
        dz = z - cz
        d = dx * dx + dy * dy + dz * dz
        distance = jnp.minimum(distance, d)
        m = jnp.max(distance, axis=1, keepdims=True)
        eq = distance == m
        eq4 = jnp.concatenate([eq, eq, eq, eq], axis=0)
        sums = jnp.sum(jnp.where(eq4, xyz3o, 0.0), axis=1, keepdims=True)
        tie = jnp.max(sums[3 * _B:4 * _B])

        def no_tie(_):
            return sums[0:_B], sums[_B:2 * _B], sums[2 * _B:3 * _B]

        def with_tie(_):
            fidx = jnp.min(jnp.where(eq, iota, _N), axis=1, keepdims=True)
            oh = iota == fidx
            oh3 = jnp.concatenate([oh, oh, oh], axis=0)
            s2 = jnp.sum(jnp.where(oh3, xyz3, 0.0), axis=1, keepdims=True)
            return s2[0:_B], s2[_B:2 * _B], s2[2 * _B:3 * _B]

        ncx, ncy, ncz = lax.cond(tie <= 1.0, no_tie, with_tie, 0)
        return distance, ncx, ncy, ncz, ax, ay, az

    _, _, _, _, ax, ay, az = lax.fori_loop(
        0, _NP, step, (dist0, c0_ref[0], c0_ref[1], c0_ref[2],
                       acc0, acc0, acc0))
    out_ref[0] = ax
    out_ref[1] = ay
    out_ref[2] = az


def _fps(xt, c0):
    return pl.pallas_call(
        _fps_body,
        out_shape=jax.ShapeDtypeStruct((_C, _B, _NP), jnp.float32),
    )(xt, c0)


def _sqdist_body(xt_ref, q_ref, r2_ref, out_ref, kst_ref):
    xm = xt_ref[0]
    q = q_ref[0]
    dot = jax.lax.dot_general(q, xm, (((1,), (0,)), ((), ())),
                              preferred_element_type=jnp.float32)
    d = -2.0 * dot
    qx = q[:, 0:1]
    qy = q[:, 1:2]
    qz = q[:, 2:3]
    qs = (qx * qx + qy * qy) + qz * qz
    x = xm[0:1, :]
    y = xm[1:2, :]
    z = xm[2:3, :]
    ns = (x * x + y * y) + z * z
    d = d + qs
    d = d + ns
    out_ref[0] = d
    r2 = r2_ref[0]
    maskf = jnp.where(d <= r2, 1.0, 0.0)
    sel_n = lax.broadcasted_iota(jnp.int32, (_N, _NK), 0) // 16
    sel_k = lax.broadcasted_iota(jnp.int32, (_N, _NK), 1)
    sel = jnp.where(sel_n <= sel_k, 1.0, 0.0)
    cum = jax.lax.dot_general(maskf, sel, (((1,), (0,)), ((), ())),
                              preferred_element_type=jnp.float32)
    nfull = jnp.sum(jnp.where(cum < float(_NS), 1.0, 0.0), axis=1,
                    keepdims=True)
    kst = jnp.minimum(nfull + 1.0, float(_NK)).astype(jnp.int32)
    kst_ref[0] = jnp.broadcast_to(kst, (_NP, 16))


def _sqdist(xt, newxyz_t, r2s):
    return pl.pallas_call(
        _sqdist_body,
        grid=(_B,),
        in_specs=[
            pl.BlockSpec((1, _C, _N), lambda b: (b, 0, 0)),
            pl.BlockSpec((1, _NP, _C), lambda b: (b, 0, 0)),
            pl.BlockSpec(memory_space=pltpu.SMEM),
        ],
        out_specs=[
            pl.BlockSpec((1, _NP, _N), lambda b: (b, 0, 0)),
            pl.BlockSpec((1, _NP, 16), lambda b: (b, 0, 0)),
        ],
        out_shape=[
            jax.ShapeDtypeStruct((_B, _NP, _N), jnp.float32),
            jax.ShapeDtypeStruct((_B, _NP, 16), jnp.int32),
        ],
    )(xt, newxyz_t, r2s)


def _group_body(dists_hbm, tbl_hbm, corr_hbm, rad2_hbm, kst_hbm, out_hbm,
                drow0, drow1, selb, idxb, rows0, rows1, corrv, rad2v,
                kstv0, kstv1, semd, semg, semo0, semo1):
    c = lax.axis_index("c")
    s = lax.axis_index("s")
    wid = s * 2 + c
    qbase = wid * _QPW
    b = qbase // _NP
    base = (b * _N).astype(jnp.int32)
    pltpu.sync_copy(corr_hbm.at[pl.ds(qbase, _QPW)], corrv)
    pltpu.sync_copy(rad2_hbm, rad2v)
    rad2 = rad2v[...]
    iota16 = lax.iota(jnp.int32, 16)
    bufs = ((drow0, kstv0, rows0, semo0), (drow1, kstv1, rows1, semo1))

    def prefetch(ci, dr, kv):
        q0 = qbase + ci * _CH
        pltpu.async_copy(dists_hbm.at[pl.ds(q0, _CH)], dr, semd)
        pltpu.async_copy(kst_hbm.at[pl.ds(q0, _CH)], kv, semd)

    prefetch(0, drow0, kstv0)
    prefetch(1, drow1, kstv1)

    def sup_body(sup, _):
        for b2, (dr, kv, rw, so) in enumerate(bufs):
            ci = sup * 2 + b2
            q0 = qbase + ci * _CH
            pltpu.make_async_copy(dists_hbm.at[pl.ds(q0, _CH)], dr,
                                  semd).wait()
            pltpu.make_async_copy(kst_hbm.at[pl.ds(q0, _CH)], kv,
                                  semd).wait()
            @pl.when(sup > 0)
            def _():
                pltpu.make_async_copy(
                    rw, out_hbm.at[pl.ds(q0 * _NS, _CH * _NS)], so).wait()

            copies = []
            for j in range(_CH):
                def body(k, cnt, j=j, dr=dr):
                    v0 = dr[j, pl.ds(k * 32, 16)]
                    v1 = dr[j, pl.ds(k * 32 + 16, 16)]
                    m0 = v0 <= rad2
                    m1 = v1 <= rad2
                    c0 = plsc.cumsum(jnp.where(m0, 1, 0))
                    c1 = plsc.cumsum(jnp.where(m1, 1, 0))
                    pc0 = plsc.all_reduce_population_count(m0)
                    pc1 = plsc.all_reduce_population_count(m1)
                    plsc.store_scatter(selb, [cnt + c0 - 1],
                                       iota16 + k * 32, mask=m0)
                    cnt1 = cnt + pc0[0]
                    plsc.store_scatter(selb, [cnt1 + c1 - 1],
                                       iota16 + (k * 32 + 16), mask=m1)
                    return cnt1 + pc1[0]

                ks = kv[j, pl.ds(0, 16)][0]
                cnt = lax.fori_loop(0, (ks + 1) >> 1, body, jnp.int32(0))
                s0 = selb[pl.ds(0, 16)]
                first = s0[0]
                s1 = selb[pl.ds(16, 16)]
                fv = jnp.zeros((16,), jnp.int32) + first
                p0 = jnp.where(iota16 < cnt, s0, fv) + base
                p1 = jnp.where(iota16 + 16 < cnt, s1, fv) + base
                idxb[j, pl.ds(0, 16)] = p0
                idxb[j, pl.ds(16, 16)] = p1
                copies.append(
                    pltpu.async_copy(tbl_hbm.at[idxb.at[j]],
                                     rw.at[pl.ds(j * _NS, _NS)], semg))
            for j in range(_CH):
                copies[j].wait()
                cv = corrv[ci * _CH + j, pl.ds(0, 16)]

                def fix_row(r, _, j=j, cv=cv, rw=rw):
                    for u in range(4):
                        rr = j * _NS + r * 4 + u
                        rw[rr, pl.ds(0, 16)] = rw[rr, pl.ds(0, 16)] - cv
                    return 0

                lax.fori_loop(0, _NS // 4, fix_row, 0)
            pltpu.async_copy(rw, out_hbm.at[pl.ds(q0 * _NS, _CH * _NS)],
                             so)

            @pl.when(sup < _NCHUNK // 2 - 1)
            def _():
                prefetch(ci + 2, dr, kv)
        return 0

    lax.fori_loop(0, _NCHUNK // 2, sup_body, 0)
    for rw, so in ((rows0, semo0), (rows1, semo1)):
        pltpu.make_async_copy(rw, out_hbm.at[pl.ds(0, _CH * _NS)],
                              so).wait()


def _group(dists, tbl, corr, rad2, ksts):
    mesh = plsc.VectorSubcoreMesh(core_axis_name="c", subcore_axis_name="s")
    f = functools.partial(
        pl.kernel,
        out_type=jax.ShapeDtypeStruct((_B * _NP * _NS, _TW), jnp.float32),
        mesh=mesh,
        compiler_params=pltpu.CompilerParams(needs_layout_passes=False),
        scratch_types=[
            pltpu.VMEM((_CH, _N), jnp.float32),
            pltpu.VMEM((_CH, _N), jnp.float32),
            pltpu.VMEM((80,), jnp.int32),
            pltpu.VMEM((_CH, _NS), jnp.int32),
            pltpu.VMEM((_CH * _NS, _TW), jnp.float32),
            pltpu.VMEM((_CH * _NS, _TW), jnp.float32),
            pltpu.VMEM((_QPW, 16), jnp.float32),
            pltpu.VMEM((16,), jnp.float32),
            pltpu.VMEM((_CH, 16), jnp.int32),
            pltpu.VMEM((_CH, 16), jnp.int32),
            pltpu.SemaphoreType.DMA,
            pltpu.SemaphoreType.DMA,
            pltpu.SemaphoreType.DMA,
            pltpu.SemaphoreType.DMA,
        ],
    )(_group_body)
    return f(dists, tbl, corr, rad2, ksts)


def kernel(xyz, points, npoint, radius, nsample):
    del nsample
    f0 = jnp.asarray(npoint, jnp.int32) - _NP
    xt = jnp.transpose(xyz, (2, 0, 1))
    c0 = jnp.transpose(xyz[jnp.arange(_B), f0], (1, 0))[:, :, None]
    newxyz_t = _fps(xt, c0)
    new_xyz = jnp.transpose(newxyz_t, (1, 2, 0))
    r2 = (radius * radius).astype(jnp.float32)
    dists, ksts = _sqdist(jnp.transpose(xt, (1, 0, 2)), new_xyz,
                          r2.reshape(1))
    tbl = jnp.pad(
        jnp.concatenate([xyz, points], axis=-1).reshape(_B * _N, _CD),
        ((0, 0), (0, _TW - _CD)))
    corr = jnp.pad(new_xyz.reshape(_B * _NP, _C), ((0, 0), (0, 16 - _C)))
    rad2 = jnp.full((16,), r2, jnp.float32)
    out = _group(dists.reshape(_B * _NP, _N), tbl, corr, rad2,
                 ksts.reshape(_B * _NP, 16))
    new_points = out[:, :_CD].reshape(_B, _NP, _NS, _CD)
    return new_xyz, new_points


# speedup vs baseline: 1.2003x; 1.0011x over previous
import functools

import jax
import jax.numpy as jnp
from jax import lax
from jax.experimental import pallas as pl
from jax.experimental.pallas import tpu as pltpu
from jax.experimental.pallas import tpu_sc as plsc

_B, _N, _C, _D = 8, 2048, 3, 64
_NP, _NS = 512, 32
_CD = _C + _D
_TW = 128
_NW = 32
_QPW = (_B * _NP) // _NW
_CH = 8
_NCHUNK = _QPW // _CH
_NK = _N // 16


def _fps_body(xt_ref, c0_ref, out_ref):
    x = xt_ref[0]
    y = xt_ref[1]
    z = xt_ref[2]
    iota = lax.broadcasted_iota(jnp.int32, (_B, _N), 1)
    iota_np = lax.broadcasted_iota(jnp.int32, (_B, _NP), 1)
    dist0 = jnp.full((_B, _N), 1e10, jnp.float32)
    acc0 = jnp.zeros((_B, _NP), jnp.float32)
    xyz3 = jnp.concatenate([x, y, z], axis=0)
    ones = jnp.ones((_B, _N), jnp.float32)
    xyz3o = jnp.concatenate([xyz3, ones], axis=0)

    def step(t, carry):
        distance, cx, cy, cz, ax, ay, az = carry
        sel = iota_np == t
        ax = jnp.where(sel, cx, ax)
        ay = jnp.where(sel, cy, ay)
        az = jnp.where(sel, cz, az)
        dx = x - cx
        dy = y - cy
        dz = z - cz
        d = dx * dx + dy * dy + dz * dz
        distance = jnp.minimum(distance, d)
        dh = jnp.maximum(distance[:, :_N // 2], distance[:, _N // 2:])
        dq = jnp.maximum(dh[:, :_N // 4], dh[:, _N // 4:])
        m = jnp.max(dq, axis=1, keepdims=True)
        eq = distance == m
        eq4 = jnp.concatenate([eq, eq, eq, eq], axis=0)
        sums = jnp.sum(jnp.where(eq4, xyz3o, 0.0), axis=1, keepdims=True)
        tie = jnp.max(sums[3 * _B:4 * _B])

        def no_tie(_):
            return sums[0:_B], sums[_B:2 * _B], sums[2 * _B:3 * _B]

        def with_tie(_):
            fidx = jnp.min(jnp.where(eq, iota, _N), axis=1, keepdims=True)
            oh = iota == fidx
            oh3 = jnp.concatenate([oh, oh, oh], axis=0)
            s2 = jnp.sum(jnp.where(oh3, xyz3, 0.0), axis=1, keepdims=True)
            return s2[0:_B], s2[_B:2 * _B], s2[2 * _B:3 * _B]

        ncx, ncy, ncz = lax.cond(tie <= 1.0, no_tie, with_tie, 0)
        return distance, ncx, ncy, ncz, ax, ay, az

    _, _, _, _, ax, ay, az = lax.fori_loop(
        0, _NP, step, (dist0, c0_ref[0], c0_ref[1], c0_ref[2],
                       acc0, acc0, acc0))
    out_ref[0] = ax
    out_ref[1] = ay
    out_ref[2] = az


def _fps(xt, c0):
    return pl.pallas_call(
        _fps_body,
        out_shape=jax.ShapeDtypeStruct((_C, _B, _NP), jnp.float32),
    )(xt, c0)


def _sqdist_body(xt_ref, q_ref, r2_ref, out_ref, kst_ref):
    xm = xt_ref[0]
    q = q_ref[0]
    dot = jax.lax.dot_general(q, xm, (((1,), (0,)), ((), ())),
                              preferred_element_type=jnp.float32)
    d = -2.0 * dot
    qx = q[:, 0:1]
    qy = q[:, 1:2]
    qz = q[:, 2:3]
    qs = (qx * qx + qy * qy) + qz * qz
    x = xm[0:1, :]
    y = xm[1:2, :]
    z = xm[2:3, :]
    ns = (x * x + y * y) + z * z
    d = d + qs
    d = d + ns
    out_ref[0] = d
    r2 = r2_ref[0]
    maskf = jnp.where(d <= r2, 1.0, 0.0)
    sel_n = lax.broadcasted_iota(jnp.int32, (_N, _NK), 0) // 16
    sel_k = lax.broadcasted_iota(jnp.int32, (_N, _NK), 1)
    sel = jnp.where(sel_n <= sel_k, 1.0, 0.0)
    cum = jax.lax.dot_general(maskf, sel, (((1,), (0,)), ((), ())),
                              preferred_element_type=jnp.float32)
    nfull = jnp.sum(jnp.where(cum < float(_NS), 1.0, 0.0), axis=1,
                    keepdims=True)
    kst = jnp.minimum(nfull + 1.0, float(_NK)).astype(jnp.int32)
    kst_ref[0] = jnp.broadcast_to(kst, (_NP, 16))


def _sqdist(xt, newxyz_t, r2s):
    return pl.pallas_call(
        _sqdist_body,
        grid=(_B,),
        in_specs=[
            pl.BlockSpec((1, _C, _N), lambda b: (b, 0, 0)),
            pl.BlockSpec((1, _NP, _C), lambda b: (b, 0, 0)),
            pl.BlockSpec(memory_space=pltpu.SMEM),
        ],
        out_specs=[
            pl.BlockSpec((1, _NP, _N), lambda b: (b, 0, 0)),
            pl.BlockSpec((1, _NP, 16), lambda b: (b, 0, 0)),
        ],
        out_shape=[
            jax.ShapeDtypeStruct((_B, _NP, _N), jnp.float32),
            jax.ShapeDtypeStruct((_B, _NP, 16), jnp.int32),
        ],
    )(xt, newxyz_t, r2s)


def _group_body(dists_hbm, tbl_hbm, corr_hbm, rad2_hbm, kst_hbm, out_hbm,
                drow0, drow1, selb, idxb, rows0, rows1, corrv, rad2v,
                kstv0, kstv1, semd, semg, semo0, semo1):
    c = lax.axis_index("c")
    s = lax.axis_index("s")
    wid = s * 2 + c
    qbase = wid * _QPW
    b = qbase // _NP
    base = (b * _N).astype(jnp.int32)
    pltpu.sync_copy(corr_hbm.at[pl.ds(qbase, _QPW)], corrv)
    pltpu.sync_copy(rad2_hbm, rad2v)
    rad2 = rad2v[...]
    iota16 = lax.iota(jnp.int32, 16)
    bufs = ((drow0, kstv0, rows0, semo0), (drow1, kstv1, rows1, semo1))

    def prefetch(ci, dr, kv):
        q0 = qbase + ci * _CH
        pltpu.async_copy(dists_hbm.at[pl.ds(q0, _CH)], dr, semd)
        pltpu.async_copy(kst_hbm.at[pl.ds(q0, _CH)], kv, semd)

    prefetch(0, drow0, kstv0)
    prefetch(1, drow1, kstv1)

    def sup_body(sup, _):
        for b2, (dr, kv, rw, so) in enumerate(bufs):
            ci = sup * 2 + b2
            q0 = qbase + ci * _CH
            pltpu.make_async_copy(dists_hbm.at[pl.ds(q0, _CH)], dr,
                                  semd).wait()
            pltpu.make_async_copy(kst_hbm.at[pl.ds(q0, _CH)], kv,
                                  semd).wait()
            @pl.when(sup > 0)
            def _():
                pltpu.make_async_copy(
                    rw, out_hbm.at[pl.ds(q0 * _NS, _CH * _NS)], so).wait()

            copies = []
            for j in range(_CH):
                def body(k, cnt, j=j, dr=dr):
                    v0 = dr[j, pl.ds(k * 32, 16)]
                    v1 = dr[j, pl.ds(k * 32 + 16, 16)]
                    m0 = v0 <= rad2
                    m1 = v1 <= rad2
                    c0 = plsc.cumsum(jnp.where(m0, 1, 0))
                    c1 = plsc.cumsum(jnp.where(m1, 1, 0))
                    pc0 = plsc.all_reduce_population_count(m0)
                    pc1 = plsc.all_reduce_population_count(m1)
                    plsc.store_scatter(selb, [cnt + c0 - 1],
                                       iota16 + k * 32, mask=m0)
                    cnt1 = cnt + pc0[0]
                    plsc.store_scatter(selb, [cnt1 + c1 - 1],
                                       iota16 + (k * 32 + 16), mask=m1)
                    return cnt1 + pc1[0]

                ks = kv[j, pl.ds(0, 16)][0]
                cnt = lax.fori_loop(0, (ks + 1) >> 1, body, jnp.int32(0))
                s0 = selb[pl.ds(0, 16)]
                first = s0[0]
                s1 = selb[pl.ds(16, 16)]
                fv = jnp.zeros((16,), jnp.int32) + first
                p0 = jnp.where(iota16 < cnt, s0, fv) + base
                p1 = jnp.where(iota16 + 16 < cnt, s1, fv) + base
                idxb[j, pl.ds(0, 16)] = p0
                idxb[j, pl.ds(16, 16)] = p1
                copies.append(
                    pltpu.async_copy(tbl_hbm.at[idxb.at[j]],
                                     rw.at[pl.ds(j * _NS, _NS)], semg))
            for j in range(_CH):
                copies[j].wait()
                cv = corrv[ci * _CH + j, pl.ds(0, 16)]

                def fix_row(r, _, j=j, cv=cv, rw=rw):
                    for u in range(4):
                        rr = j * _NS + r * 4 + u
                        rw[rr, pl.ds(0, 16)] = rw[rr, pl.ds(0, 16)] - cv
                    return 0

                lax.fori_loop(0, _NS // 4, fix_row, 0)
            pltpu.async_copy(rw, out_hbm.at[pl.ds(q0 * _NS, _CH * _NS)],
                             so)

            @pl.when(sup < _NCHUNK // 2 - 1)
            def _():
                prefetch(ci + 2, dr, kv)
        return 0

    lax.fori_loop(0, _NCHUNK // 2, sup_body, 0)
    for rw, so in ((rows0, semo0), (rows1, semo1)):
        pltpu.make_async_copy(rw, out_hbm.at[pl.ds(0, _CH * _NS)],
                              so).wait()


def _group(dists, tbl, corr, rad2, ksts):
    mesh = plsc.VectorSubcoreMesh(core_axis_name="c", subcore_axis_name="s")
    f = functools.partial(
        pl.kernel,
        out_type=jax.ShapeDtypeStruct((_B * _NP * _NS, _TW), jnp.float32),
        mesh=mesh,
        compiler_params=pltpu.CompilerParams(needs_layout_passes=False),
        scratch_types=[
            pltpu.VMEM((_CH, _N), jnp.float32),
            pltpu.VMEM((_CH, _N), jnp.float32),
            pltpu.VMEM((80,), jnp.int32),
            pltpu.VMEM((_CH, _NS), jnp.int32),
            pltpu.VMEM((_CH * _NS, _TW), jnp.float32),
            pltpu.VMEM((_CH * _NS, _TW), jnp.float32),
            pltpu.VMEM((_QPW, 16), jnp.float32),
            pltpu.VMEM((16,), jnp.float32),
            pltpu.VMEM((_CH, 16), jnp.int32),
            pltpu.VMEM((_CH, 16), jnp.int32),
            pltpu.SemaphoreType.DMA,
            pltpu.SemaphoreType.DMA,
            pltpu.SemaphoreType.DMA,
            pltpu.SemaphoreType.DMA,
        ],
    )(_group_body)
    return f(dists, tbl, corr, rad2, ksts)


def kernel(xyz, points, npoint, radius, nsample):
    del nsample
    f0 = jnp.asarray(npoint, jnp.int32) - _NP
    xt = jnp.transpose(xyz, (2, 0, 1))
    c0 = jnp.transpose(xyz[jnp.arange(_B), f0], (1, 0))[:, :, None]
    newxyz_t = _fps(xt, c0)
    new_xyz = jnp.transpose(newxyz_t, (1, 2, 0))
    r2 = (radius * radius).astype(jnp.float32)
    dists, ksts = _sqdist(jnp.transpose(xt, (1, 0, 2)), new_xyz,
                          r2.reshape(1))
    tbl = jnp.pad(
        jnp.concatenate([xyz, points], axis=-1).reshape(_B * _N, _CD),
        ((0, 0), (0, _TW - _CD)))
    corr = jnp.pad(new_xyz.reshape(_B * _NP, _C), ((0, 0), (0, 16 - _C)))
    rad2 = jnp.full((16,), r2, jnp.float32)
    out = _group(dists.reshape(_B * _NP, _N), tbl, corr, rad2,
                 ksts.reshape(_B * _NP, 16))
    new_points = out[:, :_CD].reshape(_B, _NP, _NS, _CD)
    return new_xyz, new_points
